# Initial kernel scaffold; baseline (speedup 1.0000x reference)
import jax
import jax.numpy as jnp
from jax.experimental import pallas as pl


def kernel(input_ids, lengths, token_emb, pos_emb, Wqk, Wv, Wo, ln1_g, ln1_b, W1, b1, W2, b2, ln2_g, ln2_b, Wt, bt, rot):
    raise NotImplementedError("write your pallas kernel here")


# speedup vs baseline: 11.8174x; 11.8174x over previous
import functools

import jax
import jax.numpy as jnp
from jax import lax
from jax.experimental import pallas as pl
from jax.experimental.pallas import tpu as pltpu
from jax.experimental.pallas import tpu_sc as plsc

L = 2; D = 768; H = 12; DH = 64; FF = 3072; T = 2048; TAGS = 17
NH = 8; BS = 64; NB = 32
EPS = 1e-5
NS = 96
NW = 32
SPW = NS // NW
PW = 2 * DH

def _sc_mesh():
    return dict(
        mesh=plsc.VectorSubcoreMesh(
            core_axis_name="c", subcore_axis_name="s",
            num_cores=2, num_subcores=16),
        compiler_params=pltpu.CompilerParams(needs_layout_passes=False),
    )


def _embed_body(tab_hbm, idx_hbm, out_hbm, idx_v, rows_v, sem):
    wid = lax.axis_index("s") * 2 + lax.axis_index("c")
    base = wid * (T // NW)
    pltpu.sync_copy(idx_hbm.at[pl.ds(base, T // NW)], idx_v)
    pltpu.async_copy(tab_hbm.at[idx_v], rows_v, sem).wait()
    pltpu.sync_copy(rows_v, out_hbm.at[pl.ds(base, T // NW)])


def _embed_gather(tab, ids):
    k = pl.kernel(
        _embed_body,
        out_type=jax.ShapeDtypeStruct((T, D), jnp.float32),
        scratch_types=[
            pltpu.VMEM((T // NW,), jnp.int32),
            pltpu.VMEM((T // NW, D), jnp.float32),
            pltpu.SemaphoreType.DMA,
        ],
        **_sc_mesh(),
    )
    return k(tab, ids)


def _addpos_body(e_ref, p_ref, o_ref):
    o_ref[...] = e_ref[...] + p_ref[...]


def _addpos(emb, pos):
    return pl.pallas_call(
        _addpos_body,
        out_shape=jax.ShapeDtypeStruct((T, D), jnp.float32),
    )(emb, pos)


def _ln(x, g, b):
    m = jnp.mean(x, axis=-1, keepdims=True)
    v = jnp.mean((x - m) ** 2, axis=-1, keepdims=True)
    return (x - m) / jnp.sqrt(v + EPS) * g + b


def _hash_body(x2_ref, g_ref, b_ref, wqk_ref, wv_ref, rotf_ref, qkv_ref, bkt_ref):
    y = _ln(x2_ref[...], g_ref[...], b_ref[...])
    qk = jnp.dot(y, wqk_ref[...], preferred_element_type=jnp.float32)
    v = jnp.dot(y, wv_ref[...], preferred_element_type=jnp.float32)
    bt = x2_ref.shape[0]
    qkv_ref[...] = jnp.concatenate(
        [qk.reshape(bt, H, DH), v.reshape(bt, H, DH)], axis=-1)
    rotf = rotf_ref[...]
    for h in range(H):
        p = jnp.dot(qk[:, h * DH:(h + 1) * DH], rotf,
                    preferred_element_type=jnp.float32)
        for r in range(NH):
            pr = p[:, r * 16:(r + 1) * 16]
            logits = jnp.concatenate([pr, -pr], axis=1)
            bkt_ref[h, r, :] = jnp.argmax(logits, axis=1).astype(jnp.int32)


def _hash(x2, g, b, wqk, wv, rotf):
    bt = 256
    grid = T // bt
    return pl.pallas_call(
        _hash_body,
        grid=(grid,),
        in_specs=[
            pl.BlockSpec((bt, D), lambda i: (i, 0)),
            pl.BlockSpec((1, D), lambda i: (0, 0)),
            pl.BlockSpec((1, D), lambda i: (0, 0)),
            pl.BlockSpec((D, D), lambda i: (0, 0)),
            pl.BlockSpec((D, D), lambda i: (0, 0)),
            pl.BlockSpec((DH, NH * 16), lambda i: (0, 0)),
        ],
        out_specs=[
            pl.BlockSpec((bt, H, 2 * DH), lambda i: (i, 0, 0)),
            pl.BlockSpec((H, NH, bt), lambda i: (0, 0, i)),
        ],
        out_shape=[
            jax.ShapeDtypeStruct((T, H, 2 * DH), jnp.float32),
            jax.ShapeDtypeStruct((H, NH, T), jnp.int32),
        ],
    )(x2, g, b, wqk, wv, rotf)


def _sort_body(bkt_hbm, tab_hbm, sqkv_hbm, st_hbm, stg_hbm,
               bk_f, st_f, stq_f, stg_f, hist, offs, rows_v, sem):
    wid = lax.axis_index("s") * 2 + lax.axis_index("c")
    lanes = lax.broadcasted_iota(jnp.int32, (16,), 0)
    ones = jnp.ones((16,), jnp.int32)
    span = T // 16

    for j in range(SPW):
        sid = wid * SPW + j
        h = sid // NH
        pltpu.sync_copy(bkt_hbm.at[sid], bk_f)

        for bb in range(NB):
            hist[bb, :] = jnp.zeros((16,), jnp.int32)

        def hist_step(k, _):
            bv = plsc.load_gather(bk_f, [lanes * span + k])
            plsc.addupdate_scatter(hist, [bv, lanes], ones)
            return 0
        lax.fori_loop(0, span, hist_step, 0)

        run = jnp.int32(0)
        for bb in range(NB):
            row = hist[bb, :]
            c = plsc.cumsum(row)
            offs[bb, :] = run + c - row
            run = run + jnp.sum(row)

        def scat_step(k, _):
            gidx = lanes * span + k
            bv = plsc.load_gather(bk_f, [gidx])
            pos = plsc.load_gather(offs, [bv, lanes])
            plsc.addupdate_scatter(offs, [bv, lanes], ones)
            plsc.store_scatter(st_f, [pos], gidx)
            return 0
        lax.fori_loop(0, span, scat_step, 0)

        def xform_step(k, _):
            s = st_f[pl.ds(k * 16, 16)]
            stq_f[pl.ds(k * 16, 16)] = s * H + h
            stg_f[pl.ds(k * 16, 16)] = s + sid * T
            return 0
        lax.fori_loop(0, T // 16, xform_step, 0)

        pltpu.sync_copy(st_f, st_hbm.at[sid])
        pltpu.sync_copy(stg_f, stg_hbm.at[sid])

        for c in range(T // 128):
            pltpu.async_copy(tab_hbm.at[stq_f.at[pl.ds(c * 128, 128)]],
                             rows_v, sem).wait()
            pltpu.sync_copy(rows_v, sqkv_hbm.at[sid, pl.ds(c * 128, 128)])


def _sort_gather(bkt, tab):
    k = pl.kernel(
        _sort_body,
        out_type=[
            jax.ShapeDtypeStruct((NS, T, PW), jnp.float32),
            jax.ShapeDtypeStruct((NS, T), jnp.int32),
            jax.ShapeDtypeStruct((NS, T), jnp.int32),
        ],
        scratch_types=[
            pltpu.VMEM((T,), jnp.int32),
            pltpu.VMEM((T,), jnp.int32),
            pltpu.VMEM((T,), jnp.int32),
            pltpu.VMEM((T,), jnp.int32),
            pltpu.VMEM((NB, 16), jnp.int32),
            pltpu.VMEM((NB, 16), jnp.int32),
            pltpu.VMEM((128, PW), jnp.float32),
            pltpu.SemaphoreType.DMA,
        ],
        **_sc_mesh(),
    )
    return k(bkt, tab)


def _attn_body(qv_ref, qvp_ref, t_ref, tp_ref, so_ref):
    nc = T // BS
    cur = qv_ref[0][:, :DH]
    vc = qv_ref[0][:, DH:]
    prevq = qvp_ref[0][:, :DH]
    vp = qvp_ref[0][:, DH:]

    def norm(a):
        n = jnp.sqrt(jnp.sum(a * a, axis=-1, keepdims=True))
        return a / (n + 1e-9)

    bq = cur.reshape(nc, BS, DH)
    kc = norm(cur).reshape(nc, BS, DH)
    rolled_q = jnp.concatenate([prevq[T - BS:], cur[:T - BS]], axis=0)
    kp = norm(rolled_q).reshape(nc, BS, DH)
    bk = jnp.concatenate([kc, kp], axis=1)

    rolled_v = jnp.concatenate([vp[T - BS:], vc[:T - BS]], axis=0)
    bv = jnp.concatenate([vc.reshape(nc, BS, DH),
                          rolled_v.reshape(nc, BS, DH)], axis=1)

    tq = t_ref[0]
    tp = tp_ref[0]
    tkp = jnp.concatenate([tp[nc - 1:], tq[:nc - 1]], axis=0)
    tk = jnp.concatenate([tq, tkp], axis=1)

    dots = lax.dot_general(bq, bk, (((2,), (2,)), ((0,), (0,))),
                           preferred_element_type=jnp.float32) * (DH ** -0.5)
    mask = tq[:, :, None] == tk[:, None, :]
    dots = jnp.where(mask, -5e4, dots)
    m = jnp.max(dots, axis=-1, keepdims=True)
    e = jnp.exp(dots - m)
    s = jnp.sum(e, axis=-1, keepdims=True)
    probs = e / s
    lse = m + jnp.log(s)
    bo = lax.dot_general(probs, bv, (((2,), (1,)), ((0,), (0,))),
                         preferred_element_type=jnp.float32)
    so_ref[0] = jnp.concatenate(
        [bo.reshape(T, DH), lse.reshape(T, 1),
         jnp.zeros((T, DH - 1), jnp.float32)], axis=1)


def _attention(sqkv, st3):
    def prev_idx(i):
        return (i // NH) * NH + (i % NH + NH - 1) % NH

    return pl.pallas_call(
        _attn_body,
        grid=(NS,),
        in_specs=[
            pl.BlockSpec((1, T, PW), lambda i: (i, 0, 0)),
            pl.BlockSpec((1, T, PW), lambda i: (prev_idx(i), 0, 0)),
            pl.BlockSpec((1, T // BS, BS), lambda i: (i, 0, 0)),
            pl.BlockSpec((1, T // BS, BS), lambda i: (prev_idx(i), 0, 0)),
        ],
        out_specs=pl.BlockSpec((1, T, PW), lambda i: (i, 0, 0)),
        out_shape=jax.ShapeDtypeStruct((NS, T, PW), jnp.float32),
    )(sqkv, sqkv, st3, st3)


def _scatter_body(so_hbm, stg_hbm, o_hbm, stg2d, rows_v, sem):
    wid = lax.axis_index("s") * 2 + lax.axis_index("c")
    for j in range(SPW):
        sid = wid * SPW + j
        pltpu.sync_copy(stg_hbm.at[sid], stg2d)
        for c in range(T // 128):
            pltpu.sync_copy(so_hbm.at[sid, pl.ds(c * 128, 128)], rows_v)
            pltpu.async_copy(rows_v, o_hbm.at[stg2d.at[c]], sem).wait()


def _scatter(so, stg3):
    k = pl.kernel(
        _scatter_body,
        out_type=jax.ShapeDtypeStruct((NS * T, PW), jnp.float32),
        scratch_types=[
            pltpu.VMEM((T // 128, 128), jnp.int32),
            pltpu.VMEM((128, PW), jnp.float32),
            pltpu.SemaphoreType.DMA,
        ],
        **_sc_mesh(),
    )
    return k(so, stg3)


def _combine_body(o_ref, x1_ref, wo_ref, x1n_ref):
    ox = o_ref[...]
    o = ox[..., :DH]
    lg = ox[..., DH]
    m = jnp.max(lg, axis=1, keepdims=True)
    e = jnp.exp(lg - m)
    w = e / jnp.sum(e, axis=1, keepdims=True)
    oc = jnp.sum(o * w[..., None], axis=1)
    acc = x1_ref[...]
    for h in range(H):
        acc = acc + jnp.dot(oc[h], wo_ref[h * DH:(h + 1) * DH, :],
                            preferred_element_type=jnp.float32)
    x1n_ref[...] = acc


def _combine(o, x1, wo):
    bt = 256
    return pl.pallas_call(
        _combine_body,
        grid=(T // bt,),
        in_specs=[
            pl.BlockSpec((H, NH, bt, PW), lambda i: (0, 0, i, 0)),
            pl.BlockSpec((bt, D), lambda i: (i, 0)),
            pl.BlockSpec((D, D), lambda i: (0, 0)),
        ],
        out_specs=pl.BlockSpec((bt, D), lambda i: (i, 0)),
        out_shape=jax.ShapeDtypeStruct((T, D), jnp.float32),
    )(o, x1, wo)


def _ff_body(x1_ref, x2_ref, g_ref, b_ref, w1_ref, b1_ref, w2_ref, b2_ref, o_ref):
    y = _ln(x1_ref[...], g_ref[...], b_ref[...])
    hdn = jax.nn.gelu(jnp.dot(y, w1_ref[...],
                              preferred_element_type=jnp.float32) + b1_ref[...])
    o_ref[...] = x2_ref[...] + jnp.dot(hdn, w2_ref[...],
                                       preferred_element_type=jnp.float32) + b2_ref[...]


def _ff(x1, x2, g, b, w1, b1, w2, b2):
    bt = 256
    return pl.pallas_call(
        _ff_body,
        grid=(T // bt,),
        in_specs=[
            pl.BlockSpec((bt, D), lambda i: (i, 0)),
            pl.BlockSpec((bt, D), lambda i: (i, 0)),
            pl.BlockSpec((1, D), lambda i: (0, 0)),
            pl.BlockSpec((1, D), lambda i: (0, 0)),
            pl.BlockSpec((D, FF), lambda i: (0, 0)),
            pl.BlockSpec((1, FF), lambda i: (0, 0)),
            pl.BlockSpec((FF, D), lambda i: (0, 0)),
            pl.BlockSpec((1, D), lambda i: (0, 0)),
        ],
        out_specs=pl.BlockSpec((bt, D), lambda i: (i, 0)),
        out_shape=jax.ShapeDtypeStruct((T, D), jnp.float32),
    )(x1, x2, g, b, w1, b1, w2, b2)


def _head_body(x1_ref, x2_ref, wt_ref, bt_ref, o_ref):
    hh = (x1_ref[...] + x2_ref[...]) * 0.5
    z = jnp.dot(hh, wt_ref[...], preferred_element_type=jnp.float32) + bt_ref[...]
    m = jnp.max(z, axis=-1, keepdims=True)
    e = jnp.exp(z - m)
    lse = m + jnp.log(jnp.sum(e, axis=-1, keepdims=True))
    o_ref[0] = z - lse


def _head(x1, x2, wt, btv):
    bt = 256
    return pl.pallas_call(
        _head_body,
        grid=(T // bt,),
        in_specs=[
            pl.BlockSpec((bt, D), lambda i: (i, 0)),
            pl.BlockSpec((bt, D), lambda i: (i, 0)),
            pl.BlockSpec((D, TAGS), lambda i: (0, 0)),
            pl.BlockSpec((1, TAGS), lambda i: (0, 0)),
        ],
        out_specs=pl.BlockSpec((1, bt, TAGS), lambda i: (0, i, 0)),
        out_shape=jax.ShapeDtypeStruct((1, T, TAGS), jnp.float32),
    )(x1, x2, wt, btv)


def kernel(input_ids, lengths, token_emb, pos_emb, Wqk, Wv, Wo, ln1_g, ln1_b,
           W1, b1, W2, b2, ln2_g, ln2_b, Wt, bt, rot):
    del lengths
    ids = input_ids.reshape(T).astype(jnp.int32)
    x = _addpos(_embed_gather(token_emb, ids), pos_emb)
    x1 = x
    x2 = x
    for l in range(L):
        rotf = rot[l].reshape(DH, NH * 16)
        qkv, bkt = _hash(x2, ln1_g[l].reshape(1, D), ln1_b[l].reshape(1, D),
                         Wqk[l], Wv[l], rotf)
        tab = qkv.reshape(T * H, PW)
        sqkv, st, stg = _sort_gather(bkt.reshape(NS, T), tab)
        so = _attention(sqkv, st.reshape(NS, T // BS, BS))
        o_flat = _scatter(so, stg.reshape(NS, T // 128, 128))
        x1 = _combine(o_flat.reshape(H, NH, T, PW), x1, Wo[l])
        x2 = _ff(x1, x2, ln2_g[l].reshape(1, D), ln2_b[l].reshape(1, D),
                 W1[l], b1[l].reshape(1, FF), W2[l], b2[l].reshape(1, D))
    return _head(x1, x2, Wt, bt.reshape(1, TAGS))
